# baseline (speedup 1.0000x reference)
import functools

import jax
import jax.numpy as jnp
from jax import lax
from jax.experimental import pallas as pl
from jax.experimental.pallas import tpu as pltpu
from jax.experimental.pallas import tpu_sc as plsc

D = 384
K = 8192
BK = 8192
CCW = (0.05, 0.25, 0.6)

NC = 2
NS = 16
NW = NC * NS

_DOT_DTYPE = jnp.bfloat16


def _argmin_body(ids_ref, z_ref, znorm_ref, cb_ref, cbnorm_ref,
                 idx_ref, part_ref, cbb_sc):
    r = pl.program_id(0)

    @pl.when(r == 0)
    def _():
        cbb_sc[...] = (-2.0 * cb_ref[...]).astype(_DOT_DTYPE)

    zb = z_ref[...].astype(_DOT_DTYPE)
    m2 = lax.dot_general(zb, cbb_sc[...], (((1,), (1,)), ((), ())),
                         preferred_element_type=jnp.float32)
    dist = (znorm_ref[...] + m2) + cbnorm_ref[...]
    m = jnp.min(dist, axis=1, keepdims=True)
    loc = jnp.min(jnp.where(dist == m, ids_ref[...], K),
                  axis=1, keepdims=True)
    idx_ref[...] = loc
    part_ref[...] = jnp.sum(m, keepdims=True)[None]


def _argmin_call(ids, z, znorm, cb, cbnorm, interpret=False):
    n = z.shape[0]
    bn = min(n, 1024)
    nrb = n // bn
    return pl.pallas_call(
        _argmin_body,
        grid=(nrb,),
        in_specs=[
            pl.BlockSpec((1, K), lambda r: (0, 0)),
            pl.BlockSpec((bn, D), lambda r: (r, 0)),
            pl.BlockSpec((bn, 1), lambda r: (r, 0)),
            pl.BlockSpec((K, D), lambda r: (0, 0)),
            pl.BlockSpec((1, K), lambda r: (0, 0)),
        ],
        out_specs=[
            pl.BlockSpec((bn, 1), lambda r: (r, 0)),
            pl.BlockSpec((1, 1, 1), lambda r: (r, 0, 0)),
        ],
        out_shape=[
            jax.ShapeDtypeStruct((n, 1), jnp.int32),
            jax.ShapeDtypeStruct((nrb, 1, 1), jnp.float32),
        ],
        scratch_shapes=[
            pltpu.VMEM((K, D), _DOT_DTYPE),
        ],
        interpret=interpret,
    )(ids, z, znorm, cb, cbnorm)


_GN = (1024, 4096, 4096)


@functools.lru_cache(maxsize=None)
def _make_gather3():
    mesh = plsc.VectorSubcoreMesh(core_axis_name="c", subcore_axis_name="s")
    bs = tuple(n // NW for n in _GN)

    @functools.partial(
        pl.kernel,
        mesh=mesh,
        out_type=tuple(jax.ShapeDtypeStruct((n, D), jnp.float32)
                       for n in _GN),
        scratch_types=[
            pltpu.VMEM((bs[0],), jnp.int32),
            pltpu.VMEM((bs[1],), jnp.int32),
            pltpu.VMEM((bs[2],), jnp.int32),
            pltpu.VMEM((bs[0], D), jnp.float32),
            pltpu.VMEM((bs[1], D), jnp.float32),
            pltpu.VMEM((bs[2], D), jnp.float32),
            pltpu.SemaphoreType.DMA,
        ],
    )
    def gather(cb0, i0, cb1, i1, cb2, i2, o0, o1, o2,
               iv0, iv1, iv2, rv0, rv1, rv2, sem):
        wid = lax.axis_index("s") * NC + lax.axis_index("c")
        cbs = (cb0, cb1, cb2)
        idxs = (i0, i1, i2)
        outs = (o0, o1, o2)
        ivs = (iv0, iv1, iv2)
        rvs = (rv0, rv1, rv2)
        for lvl in range(3):
            pltpu.sync_copy(idxs[lvl].at[pl.ds(wid * bs[lvl], bs[lvl])],
                            ivs[lvl])
        copies = [pltpu.async_copy(cbs[lvl].at[ivs[lvl]], rvs[lvl], sem)
                  for lvl in range(3)]
        for c in copies:
            c.wait()
        for lvl in range(3):
            pltpu.sync_copy(rvs[lvl],
                            outs[lvl].at[pl.ds(wid * bs[lvl], bs[lvl])])

    return gather


def kernel(l0, l1, l2, cb0, cb1, cb2):
    ids = jnp.arange(K, dtype=jnp.int32)[None, :]
    idxs, losses = [], []
    for i, (z, cb) in enumerate(((l0, cb0), (l1, cb1), (l2, cb2))):
        flat = z.reshape(-1, D)
        n = flat.shape[0]
        znorm = jnp.sum(flat ** 2, axis=1, keepdims=True)
        cbnorm = jnp.sum(cb ** 2, axis=1)[None, :]
        idx2d, part = _argmin_call(ids, flat, znorm, cb, cbnorm)
        idxs.append(idx2d)
        losses.append(jnp.float32(CCW[i]) * (jnp.sum(part) / jnp.float32(n * D)))
    q0, q1, q2 = _make_gather3()(cb0, idxs[0].reshape(-1),
                                 cb1, idxs[1].reshape(-1),
                                 cb2, idxs[2].reshape(-1))
    total = (losses[0] + losses[1]) + losses[2]
    return (idxs[0].reshape(l0.shape[:-1]), idxs[1].reshape(l1.shape[:-1]),
            idxs[2].reshape(l2.shape[:-1]), total,
            q0.reshape(l0.shape), q1.reshape(l1.shape), q2.reshape(l2.shape))


# speedup vs baseline: 1.0045x; 1.0045x over previous
import functools

import jax
import jax.numpy as jnp
from jax import lax
from jax.experimental import pallas as pl
from jax.experimental.pallas import tpu as pltpu
from jax.experimental.pallas import tpu_sc as plsc

D = 384
K = 8192
BK = 8192
CCW = (0.05, 0.25, 0.6)

NC = 2
NS = 16
NW = NC * NS

_DOT_DTYPE = jnp.bfloat16


def _argmin_body(ids_ref, z_ref, znorm_ref, cb_ref, cbnorm_ref,
                 idx_ref, part_ref, cbb_sc):
    r = pl.program_id(0)

    @pl.when(r == 0)
    def _():
        cbb_sc[...] = (-2.0 * cb_ref[...]).astype(_DOT_DTYPE)

    zb = z_ref[...].astype(_DOT_DTYPE)
    m2 = lax.dot_general(zb, cbb_sc[...], (((1,), (1,)), ((), ())),
                         preferred_element_type=jnp.float32)
    dist = (znorm_ref[...] + m2) + cbnorm_ref[...]
    m = jnp.min(dist, axis=1, keepdims=True)
    loc = jnp.min(jnp.where(dist == m, ids_ref[...], K),
                  axis=1, keepdims=True)
    idx_ref[...] = loc
    part_ref[...] = jnp.sum(m, keepdims=True)[None]


def _argmin_call(ids, z, znorm, cb, cbnorm, interpret=False):
    n = z.shape[0]
    bn = min(n, 1024)
    nrb = n // bn
    return pl.pallas_call(
        _argmin_body,
        grid=(nrb,),
        in_specs=[
            pl.BlockSpec((1, K), lambda r: (0, 0)),
            pl.BlockSpec((bn, D), lambda r: (r, 0)),
            pl.BlockSpec((bn, 1), lambda r: (r, 0)),
            pl.BlockSpec((K, D), lambda r: (0, 0)),
            pl.BlockSpec((1, K), lambda r: (0, 0)),
        ],
        out_specs=[
            pl.BlockSpec((bn, 1), lambda r: (r, 0)),
            pl.BlockSpec((1, 1, 1), lambda r: (r, 0, 0)),
        ],
        out_shape=[
            jax.ShapeDtypeStruct((n, 1), jnp.int32),
            jax.ShapeDtypeStruct((nrb, 1, 1), jnp.float32),
        ],
        scratch_shapes=[
            pltpu.VMEM((K, D), _DOT_DTYPE),
        ],
        interpret=interpret,
    )(ids, z, znorm, cb, cbnorm)


@functools.lru_cache(maxsize=None)
def _make_gather(n):
    b_per_w = n // NW
    mesh = plsc.VectorSubcoreMesh(core_axis_name="c", subcore_axis_name="s")

    @functools.partial(
        pl.kernel,
        mesh=mesh,
        out_type=jax.ShapeDtypeStruct((n, D), jnp.float32),
        scratch_types=[
            pltpu.VMEM((b_per_w,), jnp.int32),
            pltpu.VMEM((b_per_w, D), jnp.float32),
            pltpu.SemaphoreType.DMA,
        ],
    )
    def gather(cb_hbm, idx_hbm, out_hbm, idx_v, rows_v, sem):
        wid = lax.axis_index("s") * NC + lax.axis_index("c")
        base = wid * b_per_w
        pltpu.sync_copy(idx_hbm.at[pl.ds(base, b_per_w)], idx_v)
        pltpu.async_copy(cb_hbm.at[idx_v], rows_v, sem).wait()
        pltpu.sync_copy(rows_v, out_hbm.at[pl.ds(base, b_per_w)])

    return gather


def kernel(l0, l1, l2, cb0, cb1, cb2):
    ids = jnp.arange(K, dtype=jnp.int32)[None, :]
    out = []
    for i, (z, cb) in enumerate(((l0, cb0), (l1, cb1), (l2, cb2))):
        flat = z.reshape(-1, D)
        n = flat.shape[0]
        znorm = jnp.sum(flat ** 2, axis=1, keepdims=True)
        cbnorm = jnp.sum(cb ** 2, axis=1)[None, :]
        idx2d, part = _argmin_call(ids, flat, znorm, cb, cbnorm)
        idx = idx2d.reshape(z.shape[:-1])
        q = _make_gather(n)(cb, idx2d.reshape(-1)).reshape(z.shape)
        loss = jnp.float32(CCW[i]) * (jnp.sum(part) / jnp.float32(n * D))
        out.append((idx, loss, q))
    (idx0, loss0, q0), (idx1, loss1, q1), (idx2_, loss2, q2) = out
    total = loss0 + loss1 + loss2
    return (idx0, idx1, idx2_, total, q0, q1, q2)


# speedup vs baseline: 1.0649x; 1.0601x over previous
import functools

import jax
import jax.numpy as jnp
from jax import lax
from jax.experimental import pallas as pl
from jax.experimental.pallas import tpu as pltpu
from jax.experimental.pallas import tpu_sc as plsc

D = 384
K = 8192
BK = 8192
CCW = (0.05, 0.25, 0.6)

NC = 2
NS = 16
NW = NC * NS

_DOT_DTYPE = jnp.bfloat16


def _argmin_body(ids_ref, zb2_ref, znorm_ref, cb_ref, cbnorm_ref,
                 idx_ref, part_ref):
    cbb = cb_ref[...].astype(_DOT_DTYPE)
    m2 = lax.dot_general(zb2_ref[...], cbb, (((1,), (1,)), ((), ())),
                         preferred_element_type=jnp.float32)
    dist = (znorm_ref[...] + m2) + cbnorm_ref[...]
    m = jnp.min(dist, axis=1, keepdims=True)
    loc = jnp.min(jnp.where(dist == m, ids_ref[...], K),
                  axis=1, keepdims=True)
    idx_ref[...] = loc
    part_ref[...] = jnp.sum(m, keepdims=True)[None]


def _argmin_call(ids, zb2, znorm, cb, cbnorm, interpret=False):
    n = zb2.shape[0]
    bn = min(n, 1024)
    nrb = n // bn
    return pl.pallas_call(
        _argmin_body,
        grid=(nrb,),
        in_specs=[
            pl.BlockSpec((1, K), lambda r: (0, 0)),
            pl.BlockSpec((bn, D), lambda r: (r, 0)),
            pl.BlockSpec((bn, 1), lambda r: (r, 0)),
            pl.BlockSpec((K, D), lambda r: (0, 0)),
            pl.BlockSpec((1, K), lambda r: (0, 0)),
        ],
        out_specs=[
            pl.BlockSpec((bn, 1), lambda r: (r, 0)),
            pl.BlockSpec((1, 1, 1), lambda r: (r, 0, 0)),
        ],
        out_shape=[
            jax.ShapeDtypeStruct((n, 1), jnp.int32),
            jax.ShapeDtypeStruct((nrb, 1, 1), jnp.float32),
        ],
        interpret=interpret,
    )(ids, zb2, znorm, cb, cbnorm)


@functools.lru_cache(maxsize=None)
def _make_gather(n):
    b_per_w = n // NW
    mesh = plsc.VectorSubcoreMesh(core_axis_name="c", subcore_axis_name="s")

    @functools.partial(
        pl.kernel,
        mesh=mesh,
        out_type=jax.ShapeDtypeStruct((n, D), jnp.float32),
        scratch_types=[
            pltpu.VMEM((b_per_w,), jnp.int32),
            pltpu.VMEM((b_per_w, D), jnp.float32),
            pltpu.SemaphoreType.DMA,
        ],
    )
    def gather(cb_hbm, idx_hbm, out_hbm, idx_v, rows_v, sem):
        wid = lax.axis_index("s") * NC + lax.axis_index("c")
        base = wid * b_per_w
        pltpu.sync_copy(idx_hbm.at[pl.ds(base, b_per_w)], idx_v)
        pltpu.async_copy(cb_hbm.at[idx_v], rows_v, sem).wait()
        pltpu.sync_copy(rows_v, out_hbm.at[pl.ds(base, b_per_w)])

    return gather


def kernel(l0, l1, l2, cb0, cb1, cb2):
    ids = jnp.arange(K, dtype=jnp.int32)[None, :]
    out = []
    for i, (z, cb) in enumerate(((l0, cb0), (l1, cb1), (l2, cb2))):
        flat = z.reshape(-1, D)
        n = flat.shape[0]
        znorm = jnp.sum(flat ** 2, axis=1, keepdims=True)
        cbnorm = jnp.sum(cb ** 2, axis=1)[None, :]
        zb2 = (-2.0 * flat).astype(_DOT_DTYPE)
        idx2d, part = _argmin_call(ids, zb2, znorm, cb, cbnorm)
        idx = idx2d.reshape(z.shape[:-1])
        q = _make_gather(n)(cb, idx2d.reshape(-1)).reshape(z.shape)
        loss = jnp.float32(CCW[i]) * (jnp.sum(part) / jnp.float32(n * D))
        out.append((idx, loss, q))
    (idx0, loss0, q0), (idx1, loss1, q1), (idx2_, loss2, q2) = out
    total = loss0 + loss1 + loss2
    return (idx0, idx1, idx2_, total, q0, q1, q2)
